# baseline (speedup 1.0000x reference)
import jax
import jax.numpy as jnp
from jax.experimental import pallas as pl
from jax.experimental.pallas import tpu as pltpu

_NSRC = 4


def _body(row_ref, col_ref, o_ref, s_ref, sems):
    d, h = row_ref.shape
    w = col_ref.shape[1]
    hw = h * w
    B = o_ref.shape[0]

    p_i = jax.lax.broadcasted_iota(jnp.int32, (h, hw), 1) // w
    p_j = jax.lax.broadcasted_iota(jnp.int32, (w, hw), 1) % w
    ii = jax.lax.broadcasted_iota(jnp.int32, (h, hw), 0)
    jj = jax.lax.broadcasted_iota(jnp.int32, (w, hw), 0)
    R = (p_i == ii).astype(jnp.float32)
    C = (p_j == jj).astype(jnp.float32)
    s_ref[0] = (
        jnp.dot(row_ref[...], R, preferred_element_type=jnp.float32,
                precision=jax.lax.Precision.HIGHEST)
        + jnp.dot(col_ref[...], C, preferred_element_type=jnp.float32,
                  precision=jax.lax.Precision.HIGHEST)
    )
    reps = [pltpu.make_async_copy(s_ref.at[0], s_ref.at[k], sems.at[k])
            for k in range(1, _NSRC)]
    for r in reps:
        r.start()
    for r in reps:
        r.wait()

    K = B // 2
    vcopies = [
        pltpu.make_async_copy(s_ref.at[b % _NSRC], o_ref.at[b], sems.at[b])
        for b in range(K)
    ]
    for c in vcopies:
        c.start()
    vcopies[0].wait()
    hcopies = [
        pltpu.make_async_copy(o_ref.at[0], o_ref.at[b], sems.at[b])
        for b in range(K, B)
    ]
    for c in hcopies:
        c.start()
    for c in vcopies[1:]:
        c.wait()
    for c in hcopies:
        c.wait()


def kernel(x, mask, row_embed, col_embed):
    B = x.shape[0]
    h, w = x.shape[-2], x.shape[-1]
    d = row_embed.shape[-1]
    rowT = row_embed.T
    colT = col_embed.T
    out = pl.pallas_call(
        _body,
        in_specs=[
            pl.BlockSpec((d, h), lambda: (0, 0)),
            pl.BlockSpec((d, w), lambda: (0, 0)),
        ],
        out_specs=pl.BlockSpec(memory_space=pl.ANY),
        out_shape=jax.ShapeDtypeStruct((B, d, h * w), jnp.float32),
        scratch_shapes=[
            pltpu.VMEM((_NSRC, d, h * w), jnp.float32),
            pltpu.SemaphoreType.DMA((max(B, _NSRC),)),
        ],
    )(rowT, colT)
    return out.reshape(B, d, h, w)


# speedup vs baseline: 9.3774x; 9.3774x over previous
import jax
import jax.numpy as jnp
from jax.experimental import pallas as pl
from jax.experimental.pallas import tpu as pltpu

_NSRC = 4


def _body(row_ref, col_ref, o_ref, s_ref, sems):
    d, h = row_ref.shape
    w = col_ref.shape[1]
    hw = h * w
    B = o_ref.shape[0]

    p_i = jax.lax.broadcasted_iota(jnp.int32, (h, hw), 1) // w
    p_j = jax.lax.broadcasted_iota(jnp.int32, (w, hw), 1) % w
    ii = jax.lax.broadcasted_iota(jnp.int32, (h, hw), 0)
    jj = jax.lax.broadcasted_iota(jnp.int32, (w, hw), 0)
    R = (p_i == ii).astype(jnp.float32)
    C = (p_j == jj).astype(jnp.float32)
    s_ref[0] = (
        jnp.dot(row_ref[...], R, preferred_element_type=jnp.float32,
                precision=jax.lax.Precision.HIGHEST)
        + jnp.dot(col_ref[...], C, preferred_element_type=jnp.float32,
                  precision=jax.lax.Precision.HIGHEST)
    )
    reps = [pltpu.make_async_copy(s_ref.at[0], s_ref.at[k], sems.at[k])
            for k in range(1, _NSRC)]
    for r in reps:
        r.start()
    for r in reps:
        r.wait()

    copies = [
        pltpu.make_async_copy(s_ref.at[b % _NSRC], o_ref.at[b], sems.at[b])
        for b in range(B)
    ]
    for c in copies:
        c.start()
    for c in copies:
        c.wait()


def kernel(x, mask, row_embed, col_embed):
    B = x.shape[0]
    h, w = x.shape[-2], x.shape[-1]
    d = row_embed.shape[-1]
    rowT = row_embed.T
    colT = col_embed.T
    out = pl.pallas_call(
        _body,
        in_specs=[
            pl.BlockSpec((d, h), lambda: (0, 0)),
            pl.BlockSpec((d, w), lambda: (0, 0)),
        ],
        out_specs=pl.BlockSpec(memory_space=pl.ANY),
        out_shape=jax.ShapeDtypeStruct((B, d, h * w), jnp.float32),
        scratch_shapes=[
            pltpu.VMEM((_NSRC, d, h * w), jnp.float32),
            pltpu.SemaphoreType.DMA((max(B, _NSRC),)),
        ],
    )(rowT, colT)
    return out.reshape(B, d, h, w)
